# baseline (speedup 1.0000x reference)
import functools

import jax
import jax.numpy as jnp
from jax import lax
from jax.experimental import pallas as pl
from jax.experimental.pallas import tpu as pltpu, tpu_sc as plsc

NC = 2
NS = 16
NW = NC * NS
L = 16

N_EDGES = 320000
D = 128
EDGES_PER_W = N_EDGES // NW
CHUNK = 80
N_CHUNKS = EDGES_PER_W // CHUNK

_mesh = plsc.VectorSubcoreMesh(core_axis_name="c", subcore_axis_name="s")


@functools.partial(
    pl.kernel,
    out_type=jax.ShapeDtypeStruct((N_EDGES,), jnp.float32),
    mesh=_mesh,
    compiler_params=pltpu.CompilerParams(needs_layout_passes=False),
    scratch_types=[
        pltpu.VMEM((N_CHUNKS, CHUNK), jnp.int32),
        pltpu.VMEM((N_CHUNKS, CHUNK), jnp.int32),
        pltpu.VMEM((CHUNK, D), jnp.int32),
        pltpu.VMEM((CHUNK, D), jnp.int32),
        pltpu.VMEM((CHUNK, D), jnp.int32),
        pltpu.VMEM((CHUNK, D), jnp.int32),
        pltpu.VMEM((L * L,), jnp.float32),
        pltpu.VMEM((EDGES_PER_W,), jnp.float32),
        pltpu.SemaphoreType.DMA,
        pltpu.SemaphoreType.DMA,
    ],
)
def _edge_dot(xc_hbm, xt_hbm, idx0_hbm, idx1_hbm, out_hbm,
              idx0_v, idx1_v, rows0a, rows1a, rows0b, rows1b,
              tr_v, out_v, sem_a, sem_b):
    wid = lax.axis_index("s") * NC + lax.axis_index("c")
    lane = lax.iota(jnp.int32, L)

    pltpu.sync_copy(idx0_hbm.at[wid], idx0_v)
    pltpu.sync_copy(idx1_hbm.at[wid], idx1_v)

    bufs = ((rows0a, rows1a, sem_a), (rows0b, rows1b, sem_b))

    def issue(c, b):
        r0, r1, s = bufs[b]
        pltpu.async_copy(xc_hbm.at[idx0_v.at[c]], r0, s)
        pltpu.async_copy(xt_hbm.at[idx1_v.at[c]], r1, s)

    def wait(c, b):
        r0, r1, s = bufs[b]
        pltpu.make_async_copy(xc_hbm.at[idx0_v.at[c]], r0, s).wait()
        pltpu.make_async_copy(xt_hbm.at[idx1_v.at[c]], r1, s).wait()

    def compute(c, b):
        r0, r1, _ = bufs[b]

        @pl.loop(0, CHUNK // L)
        def _group(g):
            for j in range(L):
                e = g * L + j
                acc = None
                for k in range(D // (2 * L)):
                    a = plsc.bitcast(r0[e, pl.ds(k * L, L)], jnp.bfloat16)
                    b = plsc.bitcast(r1[e, pl.ds(k * L, L)], jnp.bfloat16)
                    p = a * b
                    lo, hi = plsc.unpack(p, format=plsc.PackFormat.INTERLEAVED)
                    s = lo + hi
                    acc = s if acc is None else acc + s
                tr_v[pl.ds(j * L, L)] = acc
            res = plsc.load_gather(tr_v, [lane * L])
            for k in range(1, L):
                res += plsc.load_gather(tr_v, [lane * L + k])
            out_v[pl.ds(c * CHUNK + g * L, L)] = res

    issue(0, 0)

    @pl.loop(0, (N_CHUNKS + 1) // 2)
    def _pair(i):
        c0 = i * 2

        @pl.when(c0 + 1 < N_CHUNKS)
        def _():
            issue(c0 + 1, 1)

        wait(c0, 0)
        compute(c0, 0)

        @pl.when(c0 + 2 < N_CHUNKS)
        def _():
            issue(c0 + 2, 0)

        @pl.when(c0 + 1 < N_CHUNKS)
        def _():
            wait(c0 + 1, 1)
            compute(c0 + 1, 1)

    pltpu.sync_copy(out_v, out_hbm.at[pl.ds(wid * EDGES_PER_W, EDGES_PER_W)])


def kernel(x_congressperson, x_ticker, edge_label_index):
    idx = edge_label_index.astype(jnp.int32).reshape(2, NW, N_CHUNKS, CHUNK)

    def to_padded_words(x):
        xb = jnp.pad(x.astype(jnp.bfloat16), ((0, 0), (0, D)))
        return lax.bitcast_convert_type(xb.reshape(-1, D, 2), jnp.int32)

    return _edge_dot(to_padded_words(x_congressperson),
                     to_padded_words(x_ticker), idx[0], idx[1])


# speedup vs baseline: 1.3898x; 1.3898x over previous
import functools

import jax
import jax.numpy as jnp
from jax import lax
from jax.experimental import pallas as pl
from jax.experimental.pallas import tpu as pltpu, tpu_sc as plsc

NC = 2
NS = 16
NW = NC * NS
L = 16

N_EDGES = 320000
N_NODES = 10000
D = 128
EDGES_PER_W = N_EDGES // NW
CHUNK = 80
N_CHUNKS = EDGES_PER_W // CHUNK

_mesh = plsc.VectorSubcoreMesh(core_axis_name="c", subcore_axis_name="s")


@functools.partial(
    pl.kernel,
    out_type=jax.ShapeDtypeStruct((N_EDGES,), jnp.float32),
    mesh=_mesh,
    compiler_params=pltpu.CompilerParams(needs_layout_passes=False),
    scratch_types=[
        pltpu.VMEM((N_CHUNKS, CHUNK), jnp.int32),
        pltpu.VMEM((N_CHUNKS, CHUNK), jnp.int32),
        pltpu.VMEM((CHUNK, D), jnp.int32),
        pltpu.VMEM((CHUNK, D), jnp.int32),
        pltpu.VMEM((CHUNK, D), jnp.int32),
        pltpu.VMEM((CHUNK, D), jnp.int32),
        pltpu.VMEM((L * L,), jnp.float32),
        pltpu.VMEM((EDGES_PER_W,), jnp.float32),
        pltpu.SemaphoreType.DMA,
        pltpu.SemaphoreType.DMA,
    ],
)
def _edge_dot(xcd_hbm, xtd_hbm, idx0_hbm, idx1_hbm, out_hbm,
              idx0_v, idx1_v, rows0a, rows1a, rows0b, rows1b,
              tr_v, out_v, sem_a, sem_b):
    wid = lax.axis_index("s") * NC + lax.axis_index("c")
    lane = lax.iota(jnp.int32, L)

    pltpu.sync_copy(idx0_hbm.at[wid], idx0_v)
    pltpu.sync_copy(idx1_hbm.at[wid], idx1_v)

    bufs = ((rows0a, rows1a, sem_a), (rows0b, rows1b, sem_b))

    def issue(c, b):
        r0, r1, s = bufs[b]
        pltpu.async_copy(xcd_hbm.at[idx0_v.at[c]], r0, s)
        pltpu.async_copy(xtd_hbm.at[idx1_v.at[c]], r1, s)

    def wait(c, b):
        r0, r1, s = bufs[b]
        pltpu.make_async_copy(xcd_hbm.at[idx0_v.at[c]], r0, s).wait()
        pltpu.make_async_copy(xtd_hbm.at[idx1_v.at[c]], r1, s).wait()

    def compute(c, b):
        r0, r1, _ = bufs[b]

        @pl.loop(0, CHUNK // L)
        def _group(g):
            for j in range(L):
                e = g * L + j
                acc = None
                for k in range(D // (2 * L)):
                    a = plsc.bitcast(r0[e, pl.ds(k * L, L)], jnp.bfloat16)
                    b_ = plsc.bitcast(r1[e, pl.ds(k * L, L)], jnp.bfloat16)
                    p = a * b_
                    lo, hi = plsc.unpack(p, format=plsc.PackFormat.INTERLEAVED)
                    s = lo + hi
                    acc = s if acc is None else acc + s
                tr_v[pl.ds(j * L, L)] = acc
            res = plsc.load_gather(tr_v, [lane * L])
            for k in range(1, L):
                res += plsc.load_gather(tr_v, [lane * L + k])
            out_v[pl.ds(c * CHUNK + g * L, L)] = res

    issue(0, 0)

    @pl.loop(0, (N_CHUNKS + 1) // 2)
    def _pair(i):
        c0 = i * 2

        @pl.when(c0 + 1 < N_CHUNKS)
        def _():
            issue(c0 + 1, 1)

        wait(c0, 0)
        compute(c0, 0)

        @pl.when(c0 + 2 < N_CHUNKS)
        def _():
            issue(c0 + 2, 0)

        @pl.when(c0 + 1 < N_CHUNKS)
        def _():
            wait(c0 + 1, 1)
            compute(c0 + 1, 1)

    pltpu.sync_copy(out_v, out_hbm.at[pl.ds(wid * EDGES_PER_W, EDGES_PER_W)])


def kernel(x_congressperson, x_ticker, edge_label_index):
    idx = edge_label_index.astype(jnp.int32).reshape(2, NW, N_CHUNKS, CHUNK)

    def to_dup(x):
        w = lax.bitcast_convert_type(
            x.astype(jnp.bfloat16).reshape(-1, D // 2, 2), jnp.int32)
        return jnp.concatenate([w, w], axis=1)

    return _edge_dot(to_dup(x_congressperson), to_dup(x_ticker),
                     idx[0], idx[1])


# speedup vs baseline: 1.7894x; 1.2876x over previous
import functools

import jax
import jax.numpy as jnp
from jax import lax
from jax.experimental import pallas as pl
from jax.experimental.pallas import tpu as pltpu, tpu_sc as plsc

NC = 2
NS = 16
NW = NC * NS
L = 16

N_EDGES = 320000
D = 128
EDGES_PER_W = N_EDGES // NW
CHUNK = 80
N_CHUNKS = EDGES_PER_W // CHUNK

_mesh = plsc.VectorSubcoreMesh(core_axis_name="c", subcore_axis_name="s")


@functools.partial(
    pl.kernel,
    out_type=jax.ShapeDtypeStruct((N_EDGES,), jnp.float32),
    mesh=_mesh,
    compiler_params=pltpu.CompilerParams(needs_layout_passes=False),
    scratch_types=[
        pltpu.VMEM((N_CHUNKS, CHUNK), jnp.int32),
        pltpu.VMEM((N_CHUNKS, CHUNK), jnp.int32),
        pltpu.VMEM((CHUNK, D), jnp.float32),
        pltpu.VMEM((CHUNK, D), jnp.float32),
        pltpu.VMEM((CHUNK, D), jnp.float32),
        pltpu.VMEM((CHUNK, D), jnp.float32),
        pltpu.VMEM((L * L,), jnp.float32),
        pltpu.VMEM((EDGES_PER_W,), jnp.float32),
        pltpu.SemaphoreType.DMA,
        pltpu.SemaphoreType.DMA,
    ],
)
def _edge_dot(xc_hbm, xt_hbm, idx0_hbm, idx1_hbm, out_hbm,
              idx0_v, idx1_v, rows0a, rows1a, rows0b, rows1b,
              tr_v, out_v, sem_a, sem_b):
    wid = lax.axis_index("s") * NC + lax.axis_index("c")
    lane = lax.iota(jnp.int32, L)

    pltpu.sync_copy(idx0_hbm.at[wid], idx0_v)
    pltpu.sync_copy(idx1_hbm.at[wid], idx1_v)

    bufs = ((rows0a, rows1a, sem_a), (rows0b, rows1b, sem_b))

    def issue(c, b):
        r0, r1, s = bufs[b]
        pltpu.async_copy(xc_hbm.at[idx0_v.at[c]], r0, s)
        pltpu.async_copy(xt_hbm.at[idx1_v.at[c]], r1, s)

    def wait(c, b):
        r0, r1, s = bufs[b]
        pltpu.make_async_copy(xc_hbm.at[idx0_v.at[c]], r0, s).wait()
        pltpu.make_async_copy(xt_hbm.at[idx1_v.at[c]], r1, s).wait()

    def compute(c, b):
        r0, r1, _ = bufs[b]

        @pl.loop(0, CHUNK // L)
        def _group(g):
            for j in range(L):
                e = g * L + j
                acc = r0[e, pl.ds(0, L)] * r1[e, pl.ds(0, L)]
                for k in range(1, D // L):
                    acc += r0[e, pl.ds(k * L, L)] * r1[e, pl.ds(k * L, L)]
                tr_v[pl.ds(j * L, L)] = acc
            res = plsc.load_gather(tr_v, [lane * L])
            for k in range(1, L):
                res += plsc.load_gather(tr_v, [lane * L + k])
            out_v[pl.ds(c * CHUNK + g * L, L)] = res

    issue(0, 0)

    @pl.loop(0, (N_CHUNKS + 1) // 2)
    def _pair(i):
        c0 = i * 2

        @pl.when(c0 + 1 < N_CHUNKS)
        def _():
            issue(c0 + 1, 1)

        wait(c0, 0)
        compute(c0, 0)

        @pl.when(c0 + 2 < N_CHUNKS)
        def _():
            issue(c0 + 2, 0)

        @pl.when(c0 + 1 < N_CHUNKS)
        def _():
            wait(c0 + 1, 1)
            compute(c0 + 1, 1)

    pltpu.sync_copy(out_v, out_hbm.at[pl.ds(wid * EDGES_PER_W, EDGES_PER_W)])


def kernel(x_congressperson, x_ticker, edge_label_index):
    idx = edge_label_index.astype(jnp.int32).reshape(2, NW, N_CHUNKS, CHUNK)
    return _edge_dot(x_congressperson, x_ticker, idx[0], idx[1])
